# baseline (speedup 1.0000x reference)
import functools

import jax
import jax.numpy as jnp
from jax import lax
from jax.experimental import pallas as pl
from jax.experimental.pallas import tpu as pltpu
from jax.experimental.pallas import tpu_sc as plsc

N = 50000
E = 1600000
OUT_CH = 32
MAX_VALUE = 2.0 * float(int(0.01 * 640 + 2) / 640)
SCALE = 1.0 / (2.0 * MAX_VALUE)
QMAX = 65535.0
CQ = SCALE / QMAX

L = 16
NC = 2
NS = 16
NW = NC * NS

N_PAD = 51200
ZR = N_PAD // NS
E_PAD = NW * 51200
EPW = E_PAD // NW
K = 3200
NCHUNK = EPW // K
GROUPS = K // L

BN = 1024


_sc_mesh = plsc.VectorSubcoreMesh(core_axis_name="c", subcore_axis_name="s")


@functools.partial(
    pl.kernel,
    out_type=jax.ShapeDtypeStruct((NC * 2 * N_PAD,), jnp.float32),
    mesh=_sc_mesh,
    scratch_types=[
        [pltpu.VMEM((K,), jnp.int32)] * 4,
        [pltpu.VMEM((K,), jnp.int32)] * 4,
        [pltpu.VMEM((K,), jnp.int32)] * 2,
        [pltpu.VMEM((K,), jnp.int32)] * 2,
        [pltpu.VMEM((K,), jnp.float32)] * 2,
        [pltpu.VMEM((K,), jnp.float32)] * 2,
        [pltpu.VMEM((K,), jnp.float32)] * 2,
        pltpu.VMEM_SHARED((N_PAD,), jnp.float32),
        pltpu.VMEM_SHARED((N_PAD,), jnp.float32),
        pltpu.VMEM_SHARED((N_PAD,), jnp.int32),
        pltpu.VMEM_SHARED((N_PAD,), jnp.float32),
        [pltpu.SemaphoreType.DMA] * 2,
        [pltpu.SemaphoreType.DMA] * 2,
        [pltpu.SemaphoreType.DMA] * 2,
    ],
)
def _sc_accumulate(pq_h, xx_h, src_h, dst_h, zero_h, out_h,
                   sidx, didx, spq_v, dpq_v, xx_v, a_v, b_v,
                   acc_a, acc_b, pq_sh, xx_sh, gsem, ssem, isem):
    c = lax.axis_index("c")
    s = lax.axis_index("s")
    wid = c * NS + s

    tsl = pl.ds(s * ZR, ZR)
    pltpu.sync_copy(zero_h, acc_a.at[tsl])
    pltpu.sync_copy(zero_h, acc_b.at[tsl])
    pltpu.sync_copy(pq_h.at[tsl], pq_sh.at[tsl])
    pltpu.sync_copy(xx_h.at[tsl], xx_sh.at[tsl])
    plsc.subcore_barrier()

    def fetch_idx(i, bi):
        base = wid * EPW + i * K
        pltpu.async_copy(src_h.at[pl.ds(base, K)], sidx[bi], isem[bi % 2])
        pltpu.async_copy(dst_h.at[pl.ds(base, K)], didx[bi], isem[bi % 2])

    def fire_gathers(bi, bf):
        for dummy in (sidx[bi], didx[bi]):
            pltpu.make_async_copy(src_h.at[pl.ds(0, K)], dummy,
                                  isem[bi % 2]).wait()
        pltpu.async_copy(pq_sh.at[sidx[bi]], spq_v[bf], gsem[bf])
        pltpu.async_copy(pq_sh.at[didx[bi]], dpq_v[bf], gsem[bf])
        pltpu.async_copy(xx_sh.at[sidx[bi]], xx_v[bf], gsem[bf])

    def drain_gather(bf):
        for dst in (xx_v[bf], a_v[bf], b_v[bf]):
            pltpu.make_async_copy(xx_h.at[pl.ds(0, K)], dst, gsem[bf]).wait()

    def drain_scatter(bf):
        pltpu.make_async_copy(xx_h.at[pl.ds(0, K)], a_v[bf], ssem[bf]).wait()
        pltpu.make_async_copy(xx_h.at[pl.ds(0, K)], b_v[bf], ssem[bf]).wait()

    mask16 = jnp.full((L,), 0xFFFF, jnp.int32)

    def compute(bi, bf):
        def grp(j, carry2):
            for u4 in range(4):
                sl = pl.ds((4 * j + u4) * L, L)
                ws = spq_v[bf][sl]
                wd = dpq_v[bf][sl]
                dxi = (wd & mask16) - (ws & mask16)
                dyi = (lax.shift_right_logical(wd, 16)
                       - lax.shift_right_logical(ws, 16))
                p = jnp.clip(dxi.astype(jnp.float32) * CQ + 0.5, 0.0, 1.0)
                q = jnp.clip(dyi.astype(jnp.float32) * CQ + 0.5, 0.0, 1.0)
                xv = xx_v[bf][sl]
                a_v[bf][sl] = xv * p
                b_v[bf][sl] = xv * q
            return carry2

        lax.fori_loop(0, GROUPS // 4, grp, None)
        pltpu.async_copy(a_v[bf], acc_a.at[didx[bi]], ssem[bf], add=True)
        pltpu.async_copy(b_v[bf], acc_b.at[didx[bi]], ssem[bf], add=True)

    fetch_idx(0, 0)
    fetch_idx(1, 1)
    fire_gathers(0, 0)
    for i in range(4):
        if i >= 2:
            drain_scatter(i % 2)
        fetch_idx(i + 2, (i + 2) % 4)
        fire_gathers((i + 1) % 4, (i + 1) % 2)
        drain_gather(i % 2)
        compute(i % 4, i % 2)

    def quad_body(t, carry):
        for u in range(4):
            i = 4 * t + u
            drain_scatter(u % 2)
            fetch_idx(jnp.minimum(i + 2, NCHUNK - 1), (u + 2) % 4)
            fire_gathers((u + 1) % 4, (u + 1) % 2)
            drain_gather(u % 2)
            compute(u % 4, u % 2)
        return carry

    lax.fori_loop(1, NCHUNK // 4, quad_body, None)
    drain_scatter(0)
    drain_scatter(1)
    drain_gather(0)
    for dummy in (sidx[1], didx[1]):
        pltpu.make_async_copy(src_h.at[pl.ds(0, K)], dummy, isem[1]).wait()

    plsc.subcore_barrier()
    base_out = c * 2 * N_PAD + s * ZR
    pltpu.sync_copy(acc_a.at[pl.ds(s * ZR, ZR)],
                    out_h.at[pl.ds(base_out, ZR)])
    pltpu.sync_copy(acc_b.at[pl.ds(s * ZR, ZR)],
                    out_h.at[pl.ds(base_out + N_PAD, ZR)])


def _tc_expand_body(ab_ref, w4_ref, o_ref):
    o_ref[...] = lax.dot_general(
        ab_ref[...], w4_ref[...],
        dimension_numbers=(((0,), (0,)), ((), ())),
        preferred_element_type=jnp.float32)


_tc_expand = pl.pallas_call(
    _tc_expand_body,
    out_shape=jax.ShapeDtypeStruct((N_PAD, OUT_CH), jnp.float32),
    grid=(N_PAD // BN,),
    in_specs=[
        pl.BlockSpec((4, BN), lambda i: (0, i)),
        pl.BlockSpec((4, OUT_CH), lambda i: (0, 0)),
    ],
    out_specs=pl.BlockSpec((BN, OUT_CH), lambda i: (i, 0)),
)


@jax.jit
def kernel(x, pos, edge_index, W):
    qx = jnp.clip(jnp.round(pos[:, 0] * QMAX), 0.0, QMAX).astype(jnp.int32)
    qy = jnp.clip(jnp.round(pos[:, 1] * QMAX), 0.0, QMAX).astype(jnp.int32)
    pq = jnp.pad(qx | (qy << 16), (0, N_PAD - N))
    xx = jnp.pad(x[:, 0], (0, N_PAD - N))
    pad_idx = jnp.full((E_PAD - E,), N, jnp.int32)
    src = jnp.concatenate([edge_index[0], pad_idx])
    dst = jnp.concatenate([edge_index[1], pad_idx])
    zeros = jnp.zeros((ZR,), jnp.float32)

    ab = _sc_accumulate(pq, xx, src, dst, zeros)
    ab4 = ab.reshape(4, N_PAD)
    w4 = jnp.concatenate([W, W], axis=0)
    out = _tc_expand(ab4, w4)
    return out[:N]


# speedup vs baseline: 1.1033x; 1.1033x over previous
import functools

import jax
import jax.numpy as jnp
from jax import lax
from jax.experimental import pallas as pl
from jax.experimental.pallas import tpu as pltpu
from jax.experimental.pallas import tpu_sc as plsc

N = 50000
E = 1600000
OUT_CH = 32
MAX_VALUE = 2.0 * float(int(0.01 * 640 + 2) / 640)
SCALE = 1.0 / (2.0 * MAX_VALUE)
QMAX = 65535.0
CQ = SCALE / QMAX

L = 16
NC = 2
NS = 16
NW = NC * NS

N_PAD = 51200
ZR = N_PAD // NS
E_PAD = NW * 51200
EPW = E_PAD // NW
K = 3200
NCHUNK = EPW // K
GROUPS = K // L

BN = 1024


_sc_mesh = plsc.VectorSubcoreMesh(core_axis_name="c", subcore_axis_name="s")


@functools.partial(
    pl.kernel,
    out_type=jax.ShapeDtypeStruct((NC * 2 * N_PAD,), jnp.float32),
    mesh=_sc_mesh,
    scratch_types=[
        [pltpu.VMEM((K,), jnp.int32)] * 4,
        [pltpu.VMEM((K,), jnp.int32)] * 4,
        [pltpu.VMEM((K,), jnp.int32)] * 2,
        [pltpu.VMEM((K,), jnp.int32)] * 2,
        [pltpu.VMEM((K,), jnp.float32)] * 2,
        [pltpu.VMEM((K,), jnp.float32)] * 2,
        [pltpu.VMEM((K,), jnp.float32)] * 2,
        pltpu.VMEM_SHARED((N_PAD,), jnp.float32),
        pltpu.VMEM_SHARED((N_PAD,), jnp.float32),
        pltpu.VMEM_SHARED((N_PAD,), jnp.int32),
        pltpu.VMEM_SHARED((N_PAD,), jnp.float32),
        [pltpu.SemaphoreType.DMA] * 2,
        [pltpu.SemaphoreType.DMA] * 2,
        [pltpu.SemaphoreType.DMA] * 2,
    ],
)
def _sc_accumulate(pq_h, xx_h, src_h, dst_h, zero_h, out_h,
                   sidx, didx, spq_v, dpq_v, xx_v, a_v, b_v,
                   acc_a, acc_b, pq_sh, xx_sh, gsem, ssem, isem):
    c = lax.axis_index("c")
    s = lax.axis_index("s")
    wid = c * NS + s

    tsl = pl.ds(s * ZR, ZR)
    pltpu.sync_copy(zero_h, acc_a.at[tsl])
    pltpu.sync_copy(zero_h, acc_b.at[tsl])
    pltpu.sync_copy(pq_h.at[tsl], pq_sh.at[tsl])
    pltpu.sync_copy(xx_h.at[tsl], xx_sh.at[tsl])
    plsc.subcore_barrier()

    def fetch_idx(i, bi):
        base = wid * EPW + i * K
        pltpu.async_copy(src_h.at[pl.ds(base, K)], sidx[bi], isem[bi % 2])
        pltpu.async_copy(dst_h.at[pl.ds(base, K)], didx[bi], isem[bi % 2])

    def fire_gathers(bi, bf):
        for dummy in (sidx[bi], didx[bi]):
            pltpu.make_async_copy(src_h.at[pl.ds(0, K)], dummy,
                                  isem[bi % 2]).wait()
        pltpu.async_copy(pq_sh.at[sidx[bi]], spq_v[bf], gsem[bf])
        pltpu.async_copy(pq_sh.at[didx[bi]], dpq_v[bf], gsem[bf])
        pltpu.async_copy(xx_sh.at[sidx[bi]], xx_v[bf], gsem[bf])

    def drain_gather(bf):
        for dst in (xx_v[bf], a_v[bf], b_v[bf]):
            pltpu.make_async_copy(xx_h.at[pl.ds(0, K)], dst, gsem[bf]).wait()

    def drain_scatter(bf):
        pltpu.make_async_copy(xx_h.at[pl.ds(0, K)], a_v[bf], ssem[bf]).wait()
        pltpu.make_async_copy(xx_h.at[pl.ds(0, K)], b_v[bf], ssem[bf]).wait()

    mask16 = jnp.full((L,), 0xFFFF, jnp.int32)

    def compute(bi, bf):
        def grp(j, carry2):
            for u4 in range(4):
                sl = pl.ds((4 * j + u4) * L, L)
                ws = spq_v[bf][sl]
                wd = dpq_v[bf][sl]
                dxi = (wd & mask16) - (ws & mask16)
                dyi = (lax.shift_right_logical(wd, 16)
                       - lax.shift_right_logical(ws, 16))
                p = jnp.clip(dxi.astype(jnp.float32) * CQ + 0.5, 0.0, 1.0)
                q = jnp.clip(dyi.astype(jnp.float32) * CQ + 0.5, 0.0, 1.0)
                xv = xx_v[bf][sl]
                a_v[bf][sl] = xv * p
                b_v[bf][sl] = xv * q
            return carry2

        lax.fori_loop(0, GROUPS // 4, grp, None)
        pltpu.async_copy(a_v[bf], acc_a.at[didx[bi]], ssem[bf], add=True)
        pltpu.async_copy(b_v[bf], acc_b.at[didx[bi]], ssem[bf], add=True)

    fetch_idx(0, 0)
    fetch_idx(1, 1)
    fire_gathers(0, 0)
    for i in range(4):
        if i >= 2:
            drain_scatter(i % 2)
        fetch_idx(i + 2, (i + 2) % 4)
        fire_gathers((i + 1) % 4, (i + 1) % 2)
        drain_gather(i % 2)
        compute(i % 4, i % 2)

    def quad_body(t, carry):
        for u in range(4):
            i = 4 * t + u
            drain_scatter(u % 2)
            fetch_idx(jnp.minimum(i + 2, NCHUNK - 1), (u + 2) % 4)
            fire_gathers((u + 1) % 4, (u + 1) % 2)
            drain_gather(u % 2)
            compute(u % 4, u % 2)
        return carry

    lax.fori_loop(1, NCHUNK // 4, quad_body, None)
    drain_scatter(0)
    drain_scatter(1)
    drain_gather(0)
    for dummy in (sidx[1], didx[1]):
        pltpu.make_async_copy(src_h.at[pl.ds(0, K)], dummy, isem[1]).wait()

    plsc.subcore_barrier()
    base_out = c * 2 * N_PAD + s * ZR
    pltpu.sync_copy(acc_a.at[pl.ds(s * ZR, ZR)],
                    out_h.at[pl.ds(base_out, ZR)])
    pltpu.sync_copy(acc_b.at[pl.ds(s * ZR, ZR)],
                    out_h.at[pl.ds(base_out + N_PAD, ZR)])


def _tc_expand_body(ab_ref, w4_ref, o_ref):
    o_ref[...] = lax.dot_general(
        ab_ref[:, :N], w4_ref[...],
        dimension_numbers=(((0,), (0,)), ((), ())),
        preferred_element_type=jnp.float32)


_tc_expand = pl.pallas_call(
    _tc_expand_body,
    out_shape=jax.ShapeDtypeStruct((N, OUT_CH), jnp.float32),
)


@jax.jit
def kernel(x, pos, edge_index, W):
    qx = jnp.clip(jnp.round(pos[:, 0] * QMAX), 0.0, QMAX).astype(jnp.int32)
    qy = jnp.clip(jnp.round(pos[:, 1] * QMAX), 0.0, QMAX).astype(jnp.int32)
    pq = jnp.pad(qx | (qy << 16), (0, N_PAD - N))
    xx = jnp.pad(x[:, 0], (0, N_PAD - N))
    pad_idx = jnp.full((E_PAD - E,), N, jnp.int32)
    src = jnp.concatenate([edge_index[0], pad_idx])
    dst = jnp.concatenate([edge_index[1], pad_idx])
    zeros = jnp.zeros((ZR,), jnp.float32)

    ab = _sc_accumulate(pq, xx, src, dst, zeros)
    ab4 = ab.reshape(4, N_PAD)
    w4 = jnp.concatenate([W, W], axis=0)
    return _tc_expand(ab4, w4)
